# baseline (speedup 1.0000x reference)
import functools

import jax
import jax.numpy as jnp
from jax import lax
from jax.experimental import pallas as pl
from jax.experimental.pallas import tpu as pltpu
from jax.experimental.pallas import tpu_sc as plsc

B = 16384
NUM_FEAT = 32
CAT_VOCAB = 1001
CAT_DIM = 16
EMB_OTHER = 64
HIDDEN = 128

_IDX_CHUNK = 128


def _sc_gather(xc0, xc1, cidx, midx, emb_pcd, emb_ct, emb_card, emb_merch):
    info = plsc.get_sparse_core_info()
    NC, NS = info.num_cores, info.num_subcores
    NW = NC * NS
    n = B // NW
    nchunk = n // _IDX_CHUNK

    mesh = plsc.VectorSubcoreMesh(core_axis_name="c", subcore_axis_name="s")

    @functools.partial(
        pl.kernel,
        mesh=mesh,
        compiler_params=pltpu.CompilerParams(use_tc_tiling_on_sc=False),
        out_type=[
            jax.ShapeDtypeStruct((B, CAT_DIM), jnp.float32),
            jax.ShapeDtypeStruct((B, CAT_DIM), jnp.float32),
            jax.ShapeDtypeStruct((B, EMB_OTHER), jnp.float32),
            jax.ShapeDtypeStruct((B, EMB_OTHER), jnp.float32),
        ],
        scratch_types=[
            pltpu.VMEM((n,), jnp.int32),
            pltpu.VMEM((n,), jnp.int32),
            pltpu.VMEM((nchunk, _IDX_CHUNK), jnp.int32),
            pltpu.VMEM((nchunk, _IDX_CHUNK), jnp.int32),
            pltpu.VMEM((nchunk, _IDX_CHUNK), jnp.int32),
            pltpu.VMEM((nchunk, _IDX_CHUNK), jnp.int32),
            pltpu.VMEM((n, CAT_DIM), jnp.float32),
            pltpu.VMEM((n, CAT_DIM), jnp.float32),
            pltpu.VMEM((n, EMB_OTHER), jnp.float32),
            pltpu.VMEM((n, EMB_OTHER), jnp.float32),
            pltpu.SemaphoreType.DMA,
        ],
    )
    def k(xc0_h, xc1_h, cidx_h, midx_h, pcd_h, ct_h, card_h, merch_h,
          e0_o, e1_o, card_o, merch_o,
          xc0_v, xc1_v, idx0_v, idx1_v, cidx_v, midx_v,
          e0_v, e1_v, card_v, merch_v, sem):
        wid = lax.axis_index("s") * NC + lax.axis_index("c")
        base = wid * n
        pltpu.sync_copy(xc0_h.at[pl.ds(base, n)], xc0_v)
        pltpu.sync_copy(xc1_h.at[pl.ds(base, n)], xc1_v)
        pltpu.sync_copy(cidx_h.at[pl.ds(wid * nchunk, nchunk)], cidx_v)
        pltpu.sync_copy(midx_h.at[pl.ds(wid * nchunk, nchunk)], midx_v)
        for j in range(nchunk):
            for t in range(_IDX_CHUNK // 16):
                off = (j * (_IDX_CHUNK // 16) + t) * 16
                v0 = xc0_v[pl.ds(off, 16)]
                v1 = xc1_v[pl.ds(off, 16)]
                idx0_v[j, pl.ds(t * 16, 16)] = jnp.clip(v0 + 1, 0, CAT_VOCAB - 1)
                idx1_v[j, pl.ds(t * 16, 16)] = jnp.clip(v1 + 1, 0, CAT_VOCAB - 1)
        copies = []
        for j in range(nchunk):
            dst = pl.ds(j * _IDX_CHUNK, _IDX_CHUNK)
            copies.append(pltpu.async_copy(card_h.at[cidx_v.at[j]], card_v.at[dst], sem))
            copies.append(pltpu.async_copy(merch_h.at[midx_v.at[j]], merch_v.at[dst], sem))
            copies.append(pltpu.async_copy(pcd_h.at[idx0_v.at[j]], e0_v.at[dst], sem))
            copies.append(pltpu.async_copy(ct_h.at[idx1_v.at[j]], e1_v.at[dst], sem))
        for c in copies:
            c.wait()
        out = pl.ds(base, n)
        pltpu.sync_copy(e0_v, e0_o.at[out])
        pltpu.sync_copy(e1_v, e1_o.at[out])
        pltpu.sync_copy(card_v, card_o.at[out])
        pltpu.sync_copy(merch_v, merch_o.at[out])

    return k(xc0, xc1, cidx, midx, emb_pcd, emb_ct, emb_card, emb_merch)


_BLK = 1024


def _tc_body(xn, e0r, e1r, cr, mr, wt, bt, wc, bc, wm, bm, to, co, mo):
    acc = jnp.dot(xn[:], wt[pl.ds(0, NUM_FEAT), :],
                  preferred_element_type=jnp.float32)
    acc += jnp.dot(e0r[:], wt[pl.ds(NUM_FEAT, CAT_DIM), :],
                   preferred_element_type=jnp.float32)
    acc += jnp.dot(e1r[:], wt[pl.ds(NUM_FEAT + CAT_DIM, CAT_DIM), :],
                   preferred_element_type=jnp.float32)
    to[:] = acc + bt[:]
    co[:] = jnp.dot(cr[:], wc[:], preferred_element_type=jnp.float32) + bc[:]
    mo[:] = jnp.dot(mr[:], wm[:], preferred_element_type=jnp.float32) + bm[:]


def _tc_forward(x_num, e0, e1, card_rows, merch_rows,
                W_trans, b_trans, W_card, b_card, W_merch, b_merch):
    grid = (B // _BLK,)
    row_blk = lambda w: pl.BlockSpec((_BLK, w), lambda i: (i, 0))
    full = lambda a: pl.BlockSpec(a.shape, lambda i: (0,) * a.ndim)
    return pl.pallas_call(
        _tc_body,
        grid=grid,
        in_specs=[
            row_blk(NUM_FEAT), row_blk(CAT_DIM), row_blk(CAT_DIM),
            row_blk(EMB_OTHER), row_blk(EMB_OTHER),
            full(W_trans), full(b_trans), full(W_card), full(b_card),
            full(W_merch), full(b_merch),
        ],
        out_specs=[row_blk(HIDDEN), row_blk(HIDDEN), row_blk(HIDDEN)],
        out_shape=[jax.ShapeDtypeStruct((B, HIDDEN), jnp.float32)] * 3,
    )(x_num, e0, e1, card_rows, merch_rows,
      W_trans, b_trans, W_card, b_card, W_merch, b_merch)


def kernel(x_num, x_cat, n_id_card, n_id_merchant,
           emb_pcd, emb_ct, W_trans, b_trans,
           emb_card, W_card, b_card,
           emb_merch, W_merch, b_merch):
    xc0 = x_cat[:, 0].astype(jnp.int32)
    xc1 = x_cat[:, 1].astype(jnp.int32)
    cidx = n_id_card.astype(jnp.int32).reshape(B // _IDX_CHUNK, _IDX_CHUNK)
    midx = n_id_merchant.astype(jnp.int32).reshape(B // _IDX_CHUNK, _IDX_CHUNK)
    e0, e1, card_rows, merch_rows = _sc_gather(
        xc0, xc1, cidx, midx, emb_pcd, emb_ct, emb_card, emb_merch)
    b_t = b_trans.reshape(1, HIDDEN)
    b_c = b_card.reshape(1, HIDDEN)
    b_m = b_merch.reshape(1, HIDDEN)
    return _tc_forward(x_num, e0, e1, card_rows, merch_rows,
                       W_trans, b_t, W_card, b_c, W_merch, b_m)


# speedup vs baseline: 1.5907x; 1.5907x over previous
import functools

import jax
import jax.numpy as jnp
from jax import lax
from jax.experimental import pallas as pl
from jax.experimental.pallas import tpu as pltpu
from jax.experimental.pallas import tpu_sc as plsc

B = 16384
NUM_FEAT = 32
CAT_VOCAB = 1001
CAT_DIM = 16
EMB_OTHER = 64
HIDDEN = 128

_IDX_CHUNK = 128


def _sc_gather(xc0, xc1, nidc, nidm, emb_pcd, emb_ct, emb_card, emb_merch):
    info = plsc.get_sparse_core_info()
    NC, NS = info.num_cores, info.num_subcores
    NW = NC * NS
    n = B // NW
    CH = 64
    nchunk = n // CH

    mesh = plsc.VectorSubcoreMesh(core_axis_name="c", subcore_axis_name="s")

    @functools.partial(
        pl.kernel,
        mesh=mesh,
        out_type=[
            jax.ShapeDtypeStruct((B, CAT_DIM), jnp.float32),
            jax.ShapeDtypeStruct((B, CAT_DIM), jnp.float32),
            jax.ShapeDtypeStruct((B, EMB_OTHER), jnp.float32),
            jax.ShapeDtypeStruct((B, EMB_OTHER), jnp.float32),
        ],
        scratch_types=[
            pltpu.VMEM((4, B // (2 * 16)), jnp.int32),
            pltpu.VMEM((2, CH, CAT_DIM), jnp.float32),
            pltpu.VMEM((2, CH, CAT_DIM), jnp.float32),
            pltpu.VMEM((2, CH, EMB_OTHER), jnp.float32),
            pltpu.VMEM((2, CH, EMB_OTHER), jnp.float32),
            pltpu.SemaphoreType.DMA,
        ],
    )
    def k(xc0_h, xc1_h, nidc_h, nidm_h, pcd_h, ct_h, card_h, merch_h,
          e0_o, e1_o, card_o, merch_o,
          idx_v, e0_b, e1_b, card_b, merch_b, sem):
        wid = lax.axis_index("s") * NC + lax.axis_index("c")
        base = wid * n
        src = pl.ds(base, n)
        pltpu.sync_copy(xc0_h.at[src], idx_v.at[0])
        pltpu.sync_copy(xc1_h.at[src], idx_v.at[1])
        pltpu.sync_copy(nidc_h.at[src], idx_v.at[2])
        pltpu.sync_copy(nidm_h.at[src], idx_v.at[3])

        def drain_and_flush(c):
            s = c % 2
            pltpu.make_async_copy(pcd_h.at[pl.ds(0, CH), :], e0_b.at[s], sem).wait()
            pltpu.make_async_copy(ct_h.at[pl.ds(0, CH), :], e1_b.at[s], sem).wait()
            pltpu.make_async_copy(card_h.at[pl.ds(0, CH), :], card_b.at[s], sem).wait()
            pltpu.make_async_copy(merch_h.at[pl.ds(0, CH), :], merch_b.at[s], sem).wait()
            out = pl.ds(base + c * CH, CH)
            pltpu.sync_copy(e0_b.at[s], e0_o.at[out])
            pltpu.sync_copy(e1_b.at[s], e1_o.at[out])
            pltpu.sync_copy(card_b.at[s], card_o.at[out])
            pltpu.sync_copy(merch_b.at[s], merch_o.at[out])

        for c in range(nchunk):
            s = c % 2

            def issue_group(g, _):
                qb = c * CH + g * 16
                v0 = jnp.clip(idx_v[0, pl.ds(qb, 16)] + 1, 0, CAT_VOCAB - 1)
                v1 = jnp.clip(idx_v[1, pl.ds(qb, 16)] + 1, 0, CAT_VOCAB - 1)
                vc = idx_v[2, pl.ds(qb, 16)]
                vm = idx_v[3, pl.ds(qb, 16)]
                for lane in range(16):
                    row = pl.ds(g * 16 + lane, 1)
                    pltpu.async_copy(pcd_h.at[pl.ds(v0[lane], 1), :], e0_b.at[s, row, :], sem)
                    pltpu.async_copy(ct_h.at[pl.ds(v1[lane], 1), :], e1_b.at[s, row, :], sem)
                    pltpu.async_copy(card_h.at[pl.ds(vc[lane], 1), :], card_b.at[s, row, :], sem)
                    pltpu.async_copy(merch_h.at[pl.ds(vm[lane], 1), :], merch_b.at[s, row, :], sem)
                return _

            lax.fori_loop(0, CH // 16, issue_group, None)
            if c > 0:
                drain_and_flush(c - 1)
        drain_and_flush(nchunk - 1)

    return k(xc0, xc1, nidc, nidm, emb_pcd, emb_ct, emb_card, emb_merch)


_BLK = 1024


def _tc_body(xn, e0r, e1r, cr, mr, wt, bt, wc, bc, wm, bm, to, co, mo):
    acc = jnp.dot(xn[:], wt[pl.ds(0, NUM_FEAT), :],
                  preferred_element_type=jnp.float32)
    acc += jnp.dot(e0r[:], wt[pl.ds(NUM_FEAT, CAT_DIM), :],
                   preferred_element_type=jnp.float32)
    acc += jnp.dot(e1r[:], wt[pl.ds(NUM_FEAT + CAT_DIM, CAT_DIM), :],
                   preferred_element_type=jnp.float32)
    to[:] = acc + bt[:]
    co[:] = jnp.dot(cr[:], wc[:], preferred_element_type=jnp.float32) + bc[:]
    mo[:] = jnp.dot(mr[:], wm[:], preferred_element_type=jnp.float32) + bm[:]


def _tc_forward(x_num, e0, e1, card_rows, merch_rows,
                W_trans, b_trans, W_card, b_card, W_merch, b_merch):
    grid = (B // _BLK,)
    row_blk = lambda w: pl.BlockSpec((_BLK, w), lambda i: (i, 0))
    full = lambda a: pl.BlockSpec(a.shape, lambda i: (0,) * a.ndim)
    return pl.pallas_call(
        _tc_body,
        grid=grid,
        in_specs=[
            row_blk(NUM_FEAT), row_blk(CAT_DIM), row_blk(CAT_DIM),
            row_blk(EMB_OTHER), row_blk(EMB_OTHER),
            full(W_trans), full(b_trans), full(W_card), full(b_card),
            full(W_merch), full(b_merch),
        ],
        out_specs=[row_blk(HIDDEN), row_blk(HIDDEN), row_blk(HIDDEN)],
        out_shape=[jax.ShapeDtypeStruct((B, HIDDEN), jnp.float32)] * 3,
    )(x_num, e0, e1, card_rows, merch_rows,
      W_trans, b_trans, W_card, b_card, W_merch, b_merch)


def kernel(x_num, x_cat, n_id_card, n_id_merchant,
           emb_pcd, emb_ct, W_trans, b_trans,
           emb_card, W_card, b_card,
           emb_merch, W_merch, b_merch):
    xc0 = x_cat[:, 0].astype(jnp.int32)
    xc1 = x_cat[:, 1].astype(jnp.int32)
    e0, e1, card_rows, merch_rows = _sc_gather(
        xc0, xc1, n_id_card.astype(jnp.int32), n_id_merchant.astype(jnp.int32),
        emb_pcd, emb_ct, emb_card, emb_merch)
    b_t = b_trans.reshape(1, HIDDEN)
    b_c = b_card.reshape(1, HIDDEN)
    b_m = b_merch.reshape(1, HIDDEN)
    return _tc_forward(x_num, e0, e1, card_rows, merch_rows,
                       W_trans, b_t, W_card, b_c, W_merch, b_m)
